# baseline (speedup 1.0000x reference)
import functools

import jax
import jax.numpy as jnp
from jax import lax
from jax.experimental import pallas as pl
from jax.experimental.pallas import tpu as pltpu
from jax.experimental.pallas import tpu_sc as plsc

_NC = 2
_NS = 16
_L = 16
_D = 16


def _matching_sc(B, NPAD):
    chunk = B // (_NC * _NS)
    tiles = chunk // _L
    mesh = plsc.VectorSubcoreMesh(
        core_axis_name="c", subcore_axis_name="s",
        num_cores=_NC, num_subcores=_NS)

    @functools.partial(
        pl.kernel,
        out_type=jax.ShapeDtypeStruct((B,), jnp.float32),
        mesh=mesh,
        scratch_types=[
            pltpu.VMEM((chunk,), jnp.int32),
            pltpu.VMEM((chunk,), jnp.int32),
            pltpu.VMEM((chunk * _D,), jnp.int32),
            pltpu.VMEM((chunk * _D,), jnp.int32),
            pltpu.VMEM((chunk * _D,), jnp.float32),
            pltpu.VMEM((chunk * _D,), jnp.float32),
            pltpu.VMEM((_D * _D,), jnp.float32),
            pltpu.VMEM((_D * _D,), jnp.float32),
            pltpu.VMEM((128,), jnp.float32),
            pltpu.VMEM((128,), jnp.float32),
            pltpu.SMEM((_D * _D,), jnp.float32),
            pltpu.SMEM((_D * _D,), jnp.float32),
            pltpu.SMEM((_D,), jnp.float32),
            pltpu.SMEM((_D,), jnp.float32),
            pltpu.VMEM((chunk,), jnp.float32),
            pltpu.SemaphoreType.DMA,
            pltpu.SemaphoreType.DMA,
        ],
        compiler_params=pltpu.CompilerParams(
            needs_layout_passes=False, use_tc_tiling_on_sc=True),
    )
    def k(uidx_h, eidx_h, utab_h, uw_h, ub_h, etab_h, ew_h, eb_h, out_h,
          uidx_v, eidx_v, uoff_v, eoff_v, urows_v, erows_v, uw_v, ew_v,
          ub_v, eb_v, uw_s, ew_s, ub_s, eb_s, out_v, sem_u, sem_e):
        wid = lax.axis_index("s") * _NC + lax.axis_index("c")
        base = wid * chunk
        pltpu.sync_copy(uidx_h.at[pl.ds(base, chunk)], uidx_v)
        pltpu.sync_copy(eidx_h.at[pl.ds(base, chunk)], eidx_v)

        def build_offsets(idx_v, off_v):
            def body(t, carry):
                iv = idx_v[pl.ds(t * _L, _L)]
                for d in range(_D):
                    off_v[pl.ds((t * _D + d) * _L, _L)] = iv + d * NPAD
                return carry
            lax.fori_loop(0, tiles, body, 0)

        build_offsets(uidx_v, uoff_v)
        build_offsets(eidx_v, eoff_v)

        cp_u = pltpu.async_copy(utab_h.at[uoff_v], urows_v, sem_u)
        cp_e = pltpu.async_copy(etab_h.at[eoff_v], erows_v, sem_e)

        pltpu.sync_copy(uw_h, uw_v)
        pltpu.sync_copy(ew_h, ew_v)
        pltpu.sync_copy(ub_h, ub_v)
        pltpu.sync_copy(eb_h, eb_v)
        for d in range(_D):
            urow = uw_v[pl.ds(d * _D, _D)]
            erow = ew_v[pl.ds(d * _D, _D)]
            for j in range(_D):
                uw_s[d * _D + j] = urow[j]
                ew_s[d * _D + j] = erow[j]
        ubv = ub_v[pl.ds(0, _D)]
        ebv = eb_v[pl.ds(0, _D)]
        for j in range(_D):
            ub_s[j] = ubv[j]
            eb_s[j] = ebv[j]
        cp_u.wait()
        cp_e.wait()

        def tower(rows_v, w_s, b_s, t):
            tT = [rows_v[pl.ds((t * _D + d) * _L, _L)] for d in range(_D)]
            res = []
            for j in range(_D):
                acc = jnp.full((_L,), b_s[j], jnp.float32)
                for d in range(_D):
                    acc = acc + w_s[d * _D + j] * tT[d]
                res.append(jnp.maximum(acc, 0.0))
            return res

        def tile_body(t, carry):
            ures = tower(urows_v, uw_s, ub_s, t)
            eres = tower(erows_v, ew_s, eb_s, t)
            out = ures[0] * eres[0]
            for j in range(1, _D):
                out = out + ures[j] * eres[j]
            out_v[pl.ds(t * _L, _L)] = out
            return carry

        lax.fori_loop(0, tiles, tile_body, 0)
        pltpu.sync_copy(out_v, out_h.at[pl.ds(base, chunk)])

    return k


def _flat_dim_major(table):
    n = table.shape[0]
    npad = -n % 128
    t = jnp.pad(table, ((0, npad), (0, 0)))
    return t.T.reshape(-1), n + npad


def kernel(user_input, event_input, user_table, user_W, user_b,
           event_table, event_W, event_b):
    B = user_input.shape[0]
    assert B % (_NC * _NS * _L) == 0 and user_table.shape[1] == _D
    uflat, NPAD = _flat_dim_major(user_table)
    eflat, _ = _flat_dim_major(event_table)
    out = _matching_sc(B, NPAD)(
        user_input.astype(jnp.int32), event_input.astype(jnp.int32),
        uflat, user_W.reshape(-1), jnp.pad(user_b, (0, 128 - _D)),
        eflat, event_W.reshape(-1), jnp.pad(event_b, (0, 128 - _D)))
    return out.reshape(B, 1)


# speedup vs baseline: 10.2932x; 10.2932x over previous
import functools

import jax
import jax.numpy as jnp
from jax import lax
from jax.experimental import pallas as pl
from jax.experimental.pallas import tpu as pltpu
from jax.experimental.pallas import tpu_sc as plsc

_NC = 2
_NS = 16
_L = 16
_D = 16


def _matching_sc(B):
    chunk = B // (_NC * _NS)
    tiles = chunk // _L
    mesh = plsc.VectorSubcoreMesh(
        core_axis_name="c", subcore_axis_name="s",
        num_cores=_NC, num_subcores=_NS)

    @functools.partial(
        pl.kernel,
        out_type=jax.ShapeDtypeStruct((B,), jnp.float32),
        mesh=mesh,
        scratch_types=[
            pltpu.VMEM((chunk,), jnp.int32),
            pltpu.VMEM((chunk,), jnp.int32),
            pltpu.VMEM((chunk * _D,), jnp.int32),
            pltpu.VMEM((chunk * _D,), jnp.int32),
            pltpu.VMEM((chunk * _D,), jnp.float32),
            pltpu.VMEM((chunk * _D,), jnp.float32),
            pltpu.VMEM((_D * _D,), jnp.float32),
            pltpu.VMEM((_D * _D,), jnp.float32),
            pltpu.VMEM((128,), jnp.float32),
            pltpu.VMEM((128,), jnp.float32),
            pltpu.SMEM((_D * _D,), jnp.float32),
            pltpu.SMEM((_D * _D,), jnp.float32),
            pltpu.SMEM((_D,), jnp.float32),
            pltpu.SMEM((_D,), jnp.float32),
            pltpu.VMEM((chunk,), jnp.float32),
            pltpu.SemaphoreType.DMA,
            pltpu.SemaphoreType.DMA,
        ],
        compiler_params=pltpu.CompilerParams(
            needs_layout_passes=False, use_tc_tiling_on_sc=True),
    )
    def k(uidx_h, eidx_h, utab_h, uw_h, ub_h, etab_h, ew_h, eb_h, out_h,
          uidx_v, eidx_v, uoff_v, eoff_v, urows_v, erows_v, uw_v, ew_v,
          ub_v, eb_v, uw_s, ew_s, ub_s, eb_s, out_v, sem_u, sem_e):
        wid = lax.axis_index("s") * _NC + lax.axis_index("c")
        base = wid * chunk
        pltpu.sync_copy(uidx_h.at[pl.ds(base, chunk)], uidx_v)
        pltpu.sync_copy(eidx_h.at[pl.ds(base, chunk)], eidx_v)

        def build_offsets(idx_v, off_v):
            def body(t, carry):
                iv = idx_v[pl.ds(t * _L, _L)]
                s = ((iv >> 7) << 11) + (iv & 127)
                for d in range(_D):
                    off_v[pl.ds((t * _D + d) * _L, _L)] = s + d * 128
                return carry
            lax.fori_loop(0, tiles, body, 0)

        build_offsets(uidx_v, uoff_v)
        build_offsets(eidx_v, eoff_v)

        cp_u = pltpu.async_copy(utab_h.at[uoff_v], urows_v, sem_u)
        cp_e = pltpu.async_copy(etab_h.at[eoff_v], erows_v, sem_e)

        pltpu.sync_copy(uw_h, uw_v)
        pltpu.sync_copy(ew_h, ew_v)
        pltpu.sync_copy(ub_h, ub_v)
        pltpu.sync_copy(eb_h, eb_v)
        for d in range(_D):
            urow = uw_v[pl.ds(d * _D, _D)]
            erow = ew_v[pl.ds(d * _D, _D)]
            for j in range(_D):
                uw_s[d * _D + j] = urow[j]
                ew_s[d * _D + j] = erow[j]
        ubv = ub_v[pl.ds(0, _D)]
        ebv = eb_v[pl.ds(0, _D)]
        for j in range(_D):
            ub_s[j] = ubv[j]
            eb_s[j] = ebv[j]
        cp_u.wait()
        cp_e.wait()

        def tower(rows_v, w_s, b_s, t):
            tT = [rows_v[pl.ds((t * _D + d) * _L, _L)] for d in range(_D)]
            res = []
            for j in range(_D):
                acc = jnp.full((_L,), b_s[j], jnp.float32)
                for d in range(_D):
                    acc = acc + w_s[d * _D + j] * tT[d]
                res.append(jnp.maximum(acc, 0.0))
            return res

        def tile_body(t, carry):
            ures = tower(urows_v, uw_s, ub_s, t)
            eres = tower(erows_v, ew_s, eb_s, t)
            out = ures[0] * eres[0]
            for j in range(1, _D):
                out = out + ures[j] * eres[j]
            out_v[pl.ds(t * _L, _L)] = out
            return carry

        lax.fori_loop(0, tiles, tile_body, 0)
        pltpu.sync_copy(out_v, out_h.at[pl.ds(base, chunk)])

    return k


def _flat_blocked(table):
    n = table.shape[0]
    npad = -n % 128
    t = jnp.pad(table, ((0, npad), (0, 0)))
    nb = (n + npad) // 128
    return t.T.reshape(2, 8, nb, 128).transpose(2, 0, 1, 3).reshape(-1)


def kernel(user_input, event_input, user_table, user_W, user_b,
           event_table, event_W, event_b):
    B = user_input.shape[0]
    assert B % (_NC * _NS * _L) == 0 and user_table.shape[1] == _D
    uflat = _flat_blocked(user_table)
    eflat = _flat_blocked(event_table)
    out = _matching_sc(B)(
        user_input.astype(jnp.int32), event_input.astype(jnp.int32),
        uflat, user_W.reshape(-1), jnp.pad(user_b, (0, 128 - _D)),
        eflat, event_W.reshape(-1), jnp.pad(event_b, (0, 128 - _D)))
    return out.reshape(B, 1)


# speedup vs baseline: 17.2774x; 1.6785x over previous
import functools

import jax
import jax.numpy as jnp
from jax import lax
from jax.experimental import pallas as pl
from jax.experimental.pallas import tpu as pltpu
from jax.experimental.pallas import tpu_sc as plsc

_NC = 2
_NS = 16
_L = 16
_D = 16


def _matching_sc(B, DTS):
    chunk = B // (_NC * _NS)
    tiles = chunk // _L
    mesh = plsc.VectorSubcoreMesh(
        core_axis_name="c", subcore_axis_name="s",
        num_cores=_NC, num_subcores=_NS)

    @functools.partial(
        pl.kernel,
        out_type=jax.ShapeDtypeStruct((B,), jnp.float32),
        mesh=mesh,
        scratch_types=[
            pltpu.VMEM((chunk,), jnp.int32),
            pltpu.VMEM((chunk,), jnp.int32),
            pltpu.VMEM((chunk * _D,), jnp.int32),
            pltpu.VMEM((chunk * _D,), jnp.int32),
            pltpu.VMEM((chunk * _D,), jnp.float32),
            pltpu.VMEM((chunk * _D,), jnp.float32),
            pltpu.VMEM((_D * _D,), jnp.float32),
            pltpu.VMEM((_D * _D,), jnp.float32),
            pltpu.VMEM((128,), jnp.float32),
            pltpu.VMEM((128,), jnp.float32),
            pltpu.SMEM((_D * _D,), jnp.float32),
            pltpu.SMEM((_D * _D,), jnp.float32),
            pltpu.SMEM((_D,), jnp.float32),
            pltpu.SMEM((_D,), jnp.float32),
            pltpu.VMEM((chunk,), jnp.float32),
            pltpu.SemaphoreType.DMA,
            pltpu.SemaphoreType.DMA,
        ],
        compiler_params=pltpu.CompilerParams(
            needs_layout_passes=False, use_tc_tiling_on_sc=True),
    )
    def k(uidx_h, eidx_h, utab_h, uw_h, ub_h, etab_h, ew_h, eb_h, out_h,
          uidx_v, eidx_v, uoff_v, eoff_v, urows_v, erows_v, uw_v, ew_v,
          ub_v, eb_v, uw_s, ew_s, ub_s, eb_s, out_v, sem_u, sem_e):
        wid = lax.axis_index("s") * _NC + lax.axis_index("c")
        base = wid * chunk
        pltpu.sync_copy(uidx_h.at[pl.ds(base, chunk)], uidx_v)
        pltpu.sync_copy(eidx_h.at[pl.ds(base, chunk)], eidx_v)

        def build_offsets(idx_v, off_v):
            def body(t, carry):
                iv = idx_v[pl.ds(t * _L, _L)]
                s = ((iv >> 7) << 10) + (iv & 127)
                for d in range(_D):
                    cst = (d // 8) * DTS + (d % 8) * 128
                    off_v[pl.ds((t * _D + d) * _L, _L)] = s + cst
                return carry
            lax.fori_loop(0, tiles, body, 0)

        build_offsets(uidx_v, uoff_v)
        build_offsets(eidx_v, eoff_v)

        cp_u = pltpu.async_copy(utab_h.at[uoff_v], urows_v, sem_u)
        cp_e = pltpu.async_copy(etab_h.at[eoff_v], erows_v, sem_e)

        pltpu.sync_copy(uw_h, uw_v)
        pltpu.sync_copy(ew_h, ew_v)
        pltpu.sync_copy(ub_h, ub_v)
        pltpu.sync_copy(eb_h, eb_v)
        for d in range(_D):
            urow = uw_v[pl.ds(d * _D, _D)]
            erow = ew_v[pl.ds(d * _D, _D)]
            for j in range(_D):
                uw_s[d * _D + j] = urow[j]
                ew_s[d * _D + j] = erow[j]
        ubv = ub_v[pl.ds(0, _D)]
        ebv = eb_v[pl.ds(0, _D)]
        for j in range(_D):
            ub_s[j] = ubv[j]
            eb_s[j] = ebv[j]
        cp_u.wait()
        cp_e.wait()

        def tower(rows_v, w_s, b_s, t):
            tT = [rows_v[pl.ds((t * _D + d) * _L, _L)] for d in range(_D)]
            res = []
            for j in range(_D):
                acc = jnp.full((_L,), b_s[j], jnp.float32)
                for d in range(_D):
                    acc = acc + w_s[d * _D + j] * tT[d]
                res.append(jnp.maximum(acc, 0.0))
            return res

        def tile_body(t, carry):
            ures = tower(urows_v, uw_s, ub_s, t)
            eres = tower(erows_v, ew_s, eb_s, t)
            out = ures[0] * eres[0]
            for j in range(1, _D):
                out = out + ures[j] * eres[j]
            out_v[pl.ds(t * _L, _L)] = out
            return carry

        lax.fori_loop(0, tiles, tile_body, 0)
        pltpu.sync_copy(out_v, out_h.at[pl.ds(base, chunk)])

    return k


def _flat_blocked(table):
    n = table.shape[0]
    npad = -n % 128
    t = jnp.pad(table, ((0, npad), (0, 0)))
    nb = (n + npad) // 128
    return t.T.reshape(2, 8, nb, 128).transpose(0, 2, 1, 3).reshape(-1), nb * 1024


def kernel(user_input, event_input, user_table, user_W, user_b,
           event_table, event_W, event_b):
    B = user_input.shape[0]
    assert B % (_NC * _NS * _L) == 0 and user_table.shape[1] == _D
    uflat, DTS = _flat_blocked(user_table)
    eflat, _ = _flat_blocked(event_table)
    out = _matching_sc(B, DTS)(
        user_input.astype(jnp.int32), event_input.astype(jnp.int32),
        uflat, user_W.reshape(-1), jnp.pad(user_b, (0, 128 - _D)),
        eflat, event_W.reshape(-1), jnp.pad(event_b, (0, 128 - _D)))
    return out.reshape(B, 1)
